# baseline (speedup 1.0000x reference)
import functools

import jax
import jax.numpy as jnp
from jax import lax
from jax.experimental import pallas as pl
from jax.experimental.pallas import tpu as pltpu
from jax.experimental.pallas import tpu_sc as plsc

_NCH = 192
_B = 4
_ROWS = _B * _NCH
_D = 224 * 224
_SPLIT = 28
_FD = _D // _SPLIT
_FROWS = _ROWS * _SPLIT
_NC = 2
_NS = 16
_NW = _NC * _NS
_FPW = _FROWS // _NW
_GRP = 16
_NGRP = _FPW // _GRP
_NBUF = 3
_NOUT = _NGRP // _NBUF


def _make_sc_shuffle():
    mesh = plsc.VectorSubcoreMesh(core_axis_name="c", subcore_axis_name="s")

    @functools.partial(
        pl.kernel,
        out_type=jax.ShapeDtypeStruct((_FROWS, _FD), jnp.float32),
        mesh=mesh,
        compiler_params=pltpu.CompilerParams(needs_layout_passes=False),
        scratch_types=[
            pltpu.VMEM((_FPW,), jnp.int32),
            pltpu.VMEM((_NBUF, _GRP, _FD), jnp.float32),
            pltpu.SemaphoreType.DMA((_NBUF,)),
            pltpu.SemaphoreType.DMA((_NBUF,)),
        ],
    )
    def shuffle(x_hbm, ids_hbm, out_hbm, idx_v, bufs, gsem, ssem):
        wid = lax.axis_index("s") * _NC + lax.axis_index("c")
        fbase = wid * _FPW
        pltpu.sync_copy(ids_hbm.at[wid], idx_v)

        def body(t3, carry):
            cps = []
            for b in range(_NBUF):
                t = t3 * _NBUF + b

                @pl.when(t3 > 0)
                def _():
                    pltpu.make_async_copy(
                        bufs.at[b], out_hbm.at[pl.ds(0, _GRP)],
                        ssem.at[b]).wait()

                idxreg = idx_v[pl.ds(t * _GRP, _GRP)]
                cps.append(
                    pltpu.async_copy(x_hbm.at[idxreg], bufs.at[b], gsem.at[b]))
            for b in range(_NBUF):
                t = t3 * _NBUF + b
                cps[b].wait()
                pltpu.async_copy(
                    bufs.at[b], out_hbm.at[pl.ds(fbase + t * _GRP, _GRP)],
                    ssem.at[b])
            return carry

        lax.fori_loop(0, _NOUT, body, 0)
        for b in range(_NBUF):
            pltpu.make_async_copy(
                bufs.at[b], out_hbm.at[pl.ds(0, _GRP)], ssem.at[b]).wait()

    return shuffle


_sc_shuffle = _make_sc_shuffle()


def kernel(x_list, objective, indices):
    x4 = x_list.reshape(_FROWS, _FD)
    src = (jnp.arange(_B, dtype=jnp.int32)[:, None] * _NCH
           + indices[None, :].astype(jnp.int32)).reshape(_ROWS)
    fine = (src[:, None] * _SPLIT
            + jnp.arange(_SPLIT, dtype=jnp.int32)[None, :]).reshape(_NW, _FPW)
    y4 = _sc_shuffle(x4, fine)
    return (y4.reshape(_B, _NCH, 224, 224), objective)


# speedup vs baseline: 3.4877x; 3.4877x over previous
import functools

import jax
import jax.numpy as jnp
from jax import lax
from jax.experimental import pallas as pl
from jax.experimental.pallas import tpu as pltpu
from jax.experimental.pallas import tpu_sc as plsc

_NCH = 192
_B = 4
_ROWS = _B * _NCH
_H = 224
_W = 224
_NC = 2
_NS = 16
_NW = _NC * _NS
_RPW = _ROWS // _NW
_IDXPAD = 32


def _make_sc_shuffle():
    mesh = plsc.VectorSubcoreMesh(core_axis_name="c", subcore_axis_name="s")

    @functools.partial(
        pl.kernel,
        out_type=jax.ShapeDtypeStruct((_ROWS, _H, _W), jnp.float32),
        mesh=mesh,
        compiler_params=pltpu.CompilerParams(needs_layout_passes=False),
        scratch_types=[
            pltpu.VMEM((_IDXPAD,), jnp.int32),
            pltpu.VMEM((1, _H, _W), jnp.float32),
            pltpu.VMEM((1, _H, _W), jnp.float32),
            pltpu.SemaphoreType.DMA,
            pltpu.SemaphoreType.DMA,
            pltpu.SemaphoreType.DMA,
            pltpu.SemaphoreType.DMA,
        ],
    )
    def shuffle(x_hbm, ids_hbm, out_hbm, idx_v, buf0, buf1, g0, g1, s0, s1):
        wid = lax.axis_index("s") * _NC + lax.axis_index("c")
        base = wid * _RPW
        pltpu.sync_copy(ids_hbm.at[wid], idx_v)
        c0 = idx_v[pl.ds(0, 16)]
        c1 = idx_v[pl.ds(16, 16)]
        bufs = (buf0, buf1)
        gsems = (g0, g1)
        ssems = (s0, s1)
        gcp = [None, None]
        for j in range(_RPW):
            b = j % 2
            rid = (c0 if j < 16 else c1)[j % 16]
            if j >= 2:
                pltpu.make_async_copy(
                    bufs[b], out_hbm.at[pl.ds(0, 1)], ssems[b]).wait()
            gcp[b] = pltpu.async_copy(
                x_hbm.at[pl.ds(rid, 1)], bufs[b], gsems[b])
            if j >= 1:
                pb = (j - 1) % 2
                gcp[pb].wait()
                pltpu.async_copy(
                    bufs[pb], out_hbm.at[pl.ds(base + j - 1, 1)], ssems[pb])
        lb = (_RPW - 1) % 2
        gcp[lb].wait()
        pltpu.async_copy(
            bufs[lb], out_hbm.at[pl.ds(base + _RPW - 1, 1)], ssems[lb])
        pltpu.make_async_copy(buf0, out_hbm.at[pl.ds(0, 1)], s0).wait()
        pltpu.make_async_copy(buf1, out_hbm.at[pl.ds(0, 1)], s1).wait()

    return shuffle


_sc_shuffle = _make_sc_shuffle()


def kernel(x_list, objective, indices):
    x3 = x_list.reshape(_ROWS, _H, _W)
    src = (jnp.arange(_B, dtype=jnp.int32)[:, None] * _NCH
           + indices[None, :].astype(jnp.int32)).reshape(_NW, _RPW)
    ids = jnp.pad(src, ((0, 0), (0, _IDXPAD - _RPW)))
    y3 = _sc_shuffle(x3, ids)
    return (y3.reshape(_B, _NCH, _H, _W), objective)
